# Initial kernel scaffold; baseline (speedup 1.0000x reference)
import jax
import jax.numpy as jnp
from jax.experimental import pallas as pl


def kernel(edge_attr, W0, W1, W2):
    raise NotImplementedError("write your pallas kernel here")


# speedup vs baseline: 2.8335x; 2.8335x over previous
import functools

import jax
import jax.numpy as jnp
from jax import lax
from jax.experimental import pallas as pl
from jax.experimental.pallas import tpu as pltpu
from jax.experimental.pallas import tpu_sc as plsc

E = 320000
D = 128
NC = 2
NS = 16
NW = NC * NS
ROWS_PER_W = E // NW
CB = 400
NCHUNK = ROWS_PER_W // CB

_mesh = plsc.VectorSubcoreMesh(core_axis_name="c", subcore_axis_name="s")


@functools.partial(
    pl.kernel,
    out_type=jax.ShapeDtypeStruct((E, D), jnp.float32),
    mesh=_mesh,
    scratch_types=[
        pltpu.VMEM((5, D), jnp.float32),
        pltpu.VMEM((6, D), jnp.float32),
        pltpu.VMEM((2, D), jnp.float32),
        pltpu.VMEM((60, D), jnp.float32),
        pltpu.VMEM((3 * CB + 16,), jnp.int32),
        pltpu.VMEM((CB, D), jnp.float32),
    ],
)
def _bond_encoder_sc(attr_hbm, w0_hbm, w1_hbm, w2_hbm, out_hbm,
                     w0_v, w1_v, w2_v, combo_v, attr_v, out_v):
    wid = lax.axis_index("s") * NC + lax.axis_index("c")
    base = wid * ROWS_PER_W

    pltpu.sync_copy(w0_hbm, w0_v)
    pltpu.sync_copy(w1_hbm, w1_v)
    pltpu.sync_copy(w2_hbm, w2_v)

    def build_combo(c, _):
        i = c // 12
        r = c - i * 12
        j = r // 2
        k = r - j * 2
        for v in range(8):
            sl = pl.ds(16 * v, 16)
            combo_v[c, sl] = w0_v[i, sl] + w1_v[j, sl] + w2_v[k, sl]
        return _

    lax.fori_loop(0, 60, build_combo, None)

    def do_chunk(g, _):
        off = base + g * CB
        pltpu.sync_copy(attr_hbm.at[pl.ds(3 * off, 3 * CB)],
                        attr_v.at[pl.ds(0, 3 * CB)])

        def do_edge(e, _2):
            av = attr_v[pl.ds(3 * e, 16)]
            c = av[0] * 12 + av[1] * 2 + av[2]
            for v in range(8):
                sl = pl.ds(16 * v, 16)
                out_v[e, sl] = combo_v[c, sl]
            return _2

        lax.fori_loop(0, CB, do_edge, None)
        pltpu.sync_copy(out_v, out_hbm.at[pl.ds(off, CB)])
        return _

    lax.fori_loop(0, NCHUNK, do_chunk, None)


def kernel(edge_attr, W0, W1, W2):
    attr_flat = edge_attr.astype(jnp.int32).reshape(-1)
    return _bond_encoder_sc(attr_flat, W0, W1, W2)


# speedup vs baseline: 7.3834x; 2.6058x over previous
import functools

import jax
import jax.numpy as jnp
from jax import lax
from jax.experimental import pallas as pl
from jax.experimental.pallas import tpu as pltpu
from jax.experimental.pallas import tpu_sc as plsc

E = 320000
D = 128
NC = 2
NS = 16
NW = NC * NS
ROWS_PER_W = E // NW
CB = 400
NCHUNK = ROWS_PER_W // CB
APAD = 3 * CB + 16

_mesh = plsc.VectorSubcoreMesh(core_axis_name="c", subcore_axis_name="s")


@functools.partial(
    pl.kernel,
    out_type=jax.ShapeDtypeStruct((E, D), jnp.float32),
    mesh=_mesh,
    scratch_types=[
        pltpu.VMEM((5, D), jnp.float32),
        pltpu.VMEM((6, D), jnp.float32),
        pltpu.VMEM((2, D), jnp.float32),
        pltpu.VMEM((60, D), jnp.float32),
        pltpu.VMEM((APAD,), jnp.int32),
        pltpu.VMEM((APAD,), jnp.int32),
        pltpu.VMEM((2, CB, D), jnp.float32),
        pltpu.SemaphoreType.DMA,
        pltpu.SemaphoreType.DMA,
        pltpu.SemaphoreType.DMA,
        pltpu.SemaphoreType.DMA,
    ],
)
def _bond_encoder_sc(attr_hbm, w0_hbm, w1_hbm, w2_hbm, out_hbm,
                     w0_v, w1_v, w2_v, combo_v, attr_a, attr_b, out_v,
                     si0, si1, so0, so1):
    wid = lax.axis_index("s") * NC + lax.axis_index("c")
    base = wid * ROWS_PER_W
    attrs = (attr_a, attr_b)
    sin = (si0, si1)
    sout = (so0, so1)

    pltpu.sync_copy(w0_hbm, w0_v)
    pltpu.sync_copy(w1_hbm, w1_v)
    pltpu.sync_copy(w2_hbm, w2_v)

    def build_combo(c, _):
        i = c // 12
        r = c - i * 12
        j = r // 2
        k = r - j * 2
        for v in range(8):
            sl = pl.ds(16 * v, 16)
            combo_v[c, sl] = w0_v[i, sl] + w1_v[j, sl] + w2_v[k, sl]
        return _

    lax.fori_loop(0, 60, build_combo, None)

    def start_in(g, b):
        pltpu.async_copy(attr_hbm.at[pl.ds(3 * (base + g * CB), 3 * CB)],
                         attrs[b].at[pl.ds(0, 3 * CB)], sin[b])

    def wait_in(b):
        pltpu.make_async_copy(attr_hbm.at[pl.ds(0, 3 * CB)],
                              attrs[b].at[pl.ds(0, 3 * CB)], sin[b]).wait()

    def start_out(g, b):
        pltpu.async_copy(out_v.at[b], out_hbm.at[pl.ds(base + g * CB, CB)],
                         sout[b])

    def wait_out(b):
        pltpu.make_async_copy(out_v.at[b], out_hbm.at[pl.ds(0, CB)],
                              sout[b]).wait()

    def compute(b):
        @plsc.parallel_loop(0, CB // 5)
        def _edges(i):
            av = attrs[b][pl.ds(15 * i, 16)]
            for k in range(5):
                c = av[3 * k] * 12 + av[3 * k + 1] * 2 + av[3 * k + 2]
                row = 5 * i + k
                for v in range(8):
                    sl = pl.ds(16 * v, 16)
                    out_v[b, row, sl] = combo_v[c, sl]

    start_in(0, 0)
    start_in(1, 1)

    def pair_body(p, _):
        for b in (0, 1):
            g = 2 * p + b
            wait_in(b)

            @pl.when(p >= 1)
            def _():
                wait_out(b)

            compute(b)
            start_out(g, b)
            if b == 0:
                start_in(g + 2, b)
            else:
                @pl.when(p < 11)
                def _():
                    start_in(g + 2, b)
        return _

    lax.fori_loop(0, (NCHUNK - 1) // 2, pair_body, None)

    wait_in(0)
    wait_out(0)
    compute(0)
    start_out(NCHUNK - 1, 0)
    wait_out(1)
    wait_out(0)


def kernel(edge_attr, W0, W1, W2):
    attr_flat = edge_attr.astype(jnp.int32).reshape(-1)
    return _bond_encoder_sc(attr_flat, W0, W1, W2)


# speedup vs baseline: 18.8905x; 2.5585x over previous
import functools

import jax
import jax.numpy as jnp
from jax import lax
from jax.experimental import pallas as pl
from jax.experimental.pallas import tpu as pltpu
from jax.experimental.pallas import tpu_sc as plsc

E = 320000
D = 128
NC = 2
NS = 16
NW = NC * NS
ROWS_PER_W = E // NW
CB = 400
NCHUNK = ROWS_PER_W // CB

_mesh = plsc.VectorSubcoreMesh(core_axis_name="c", subcore_axis_name="s")


@functools.partial(
    pl.kernel,
    out_type=jax.ShapeDtypeStruct((E, D), jnp.float32),
    mesh=_mesh,
    scratch_types=[
        pltpu.VMEM((5, D), jnp.float32),
        pltpu.VMEM((6, D), jnp.float32),
        pltpu.VMEM((2, D), jnp.float32),
        pltpu.VMEM((60, D), jnp.float32),
        pltpu.VMEM_SHARED((60, D), jnp.float32),
        pltpu.VMEM((CB,), jnp.int32),
        pltpu.VMEM((CB,), jnp.int32),
        pltpu.VMEM((CB,), jnp.int32),
        pltpu.VMEM((CB,), jnp.int32),
        pltpu.VMEM((CB,), jnp.int32),
        pltpu.VMEM((CB,), jnp.int32),
        pltpu.VMEM((CB,), jnp.int32),
        pltpu.VMEM((CB,), jnp.int32),
        pltpu.VMEM((2, CB, D), jnp.float32),
        pltpu.SemaphoreType.DMA,
        pltpu.SemaphoreType.DMA,
        pltpu.SemaphoreType.DMA,
        pltpu.SemaphoreType.DMA,
        pltpu.SemaphoreType.DMA,
    ],
)
def _bond_encoder_sc(a0_hbm, a1_hbm, a2_hbm, w0_hbm, w1_hbm, w2_hbm, out_hbm,
                     w0_v, w1_v, w2_v, combo_v, combo_sh,
                     a00, a01, a10, a11, a20, a21, idx_a, idx_b, out_v,
                     si0, si1, so0, so1, sg):
    sid = lax.axis_index("s")
    wid = sid * NC + lax.axis_index("c")
    base = wid * ROWS_PER_W
    cols = ((a0_hbm, (a00, a01)), (a1_hbm, (a10, a11)), (a2_hbm, (a20, a21)))
    idxs = (idx_a, idx_b)
    sin = (si0, si1)
    sout = (so0, so1)

    @pl.when(sid == 0)
    def _build():
        pltpu.sync_copy(w0_hbm, w0_v)
        pltpu.sync_copy(w1_hbm, w1_v)
        pltpu.sync_copy(w2_hbm, w2_v)

        def build_combo(c, _):
            i = c // 12
            r = c - i * 12
            j = r // 2
            k = r - j * 2
            for v in range(8):
                sl = pl.ds(16 * v, 16)
                combo_v[c, sl] = w0_v[i, sl] + w1_v[j, sl] + w2_v[k, sl]
            return _

        lax.fori_loop(0, 60, build_combo, None)
        pltpu.sync_copy(combo_v, combo_sh)

    plsc.subcore_barrier()

    def start_in(g, b):
        for hbm, v in cols:
            pltpu.async_copy(hbm.at[pl.ds(base + g * CB, CB)],
                             v[b], sin[b])

    def wait_in(b):
        for hbm, v in cols:
            pltpu.make_async_copy(hbm.at[pl.ds(0, CB)], v[b],
                                  sin[b]).wait()

    def start_out(g, b):
        pltpu.async_copy(out_v.at[b], out_hbm.at[pl.ds(base + g * CB, CB)],
                         sout[b])

    def wait_out(b):
        pltpu.make_async_copy(out_v.at[b], out_hbm.at[pl.ds(0, CB)],
                              sout[b]).wait()

    def compute(b):
        @plsc.parallel_loop(0, CB // 16)
        def _t(t):
            sl = pl.ds(16 * t, 16)
            idxs[b][sl] = (cols[0][1][b][sl] * 12 + cols[1][1][b][sl] * 2
                           + cols[2][1][b][sl])

        pltpu.async_copy(combo_sh.at[idxs[b]], out_v.at[b], sg).wait()

    start_in(0, 0)
    start_in(1, 1)

    def pair_body(p, _):
        for b in (0, 1):
            g = 2 * p + b
            wait_in(b)

            @pl.when(p >= 1)
            def _():
                wait_out(b)

            compute(b)
            start_out(g, b)
            if b == 0:
                start_in(g + 2, b)
            else:
                @pl.when(p < 11)
                def _():
                    start_in(g + 2, b)
        return _

    lax.fori_loop(0, (NCHUNK - 1) // 2, pair_body, None)

    wait_in(0)
    wait_out(0)
    compute(0)
    start_out(NCHUNK - 1, 0)
    wait_out(1)
    wait_out(0)


def kernel(edge_attr, W0, W1, W2):
    ea = edge_attr.astype(jnp.int32)
    return _bond_encoder_sc(ea[:, 0], ea[:, 1], ea[:, 2], W0, W1, W2)


# speedup vs baseline: 18.8938x; 1.0002x over previous
import functools

import jax
import jax.numpy as jnp
from jax import lax
from jax.experimental import pallas as pl
from jax.experimental.pallas import tpu as pltpu
from jax.experimental.pallas import tpu_sc as plsc

E = 320000
D = 128
NC = 2
NS = 16
NW = NC * NS
ROWS_PER_W = E // NW
CB = 400
NCHUNK = ROWS_PER_W // CB

_mesh = plsc.VectorSubcoreMesh(core_axis_name="c", subcore_axis_name="s")


@functools.partial(
    pl.kernel,
    out_type=jax.ShapeDtypeStruct((E, D), jnp.float32),
    mesh=_mesh,
    scratch_types=[
        pltpu.VMEM((5, D), jnp.float32),
        pltpu.VMEM((6, D), jnp.float32),
        pltpu.VMEM((2, D), jnp.float32),
        pltpu.VMEM((60, D), jnp.float32),
        pltpu.VMEM_SHARED((60, D), jnp.float32),
        pltpu.VMEM((CB,), jnp.int32),
        pltpu.VMEM((CB,), jnp.int32),
        pltpu.VMEM((CB,), jnp.int32),
        pltpu.VMEM((CB,), jnp.int32),
        pltpu.VMEM((CB,), jnp.int32),
        pltpu.VMEM((CB,), jnp.int32),
        pltpu.VMEM((CB,), jnp.int32),
        pltpu.VMEM((CB,), jnp.int32),
        pltpu.VMEM((2, CB, D), jnp.float32),
        pltpu.SemaphoreType.DMA,
        pltpu.SemaphoreType.DMA,
        pltpu.SemaphoreType.DMA,
        pltpu.SemaphoreType.DMA,
        pltpu.SemaphoreType.DMA,
        pltpu.SemaphoreType.DMA,
    ],
)
def _bond_encoder_sc(a0_hbm, a1_hbm, a2_hbm, w0_hbm, w1_hbm, w2_hbm, out_hbm,
                     w0_v, w1_v, w2_v, combo_v, combo_sh,
                     a00, a01, a10, a11, a20, a21, idx_a, idx_b, out_v,
                     si0, si1, so0, so1, sg0, sg1):
    sid = lax.axis_index("s")
    wid = sid * NC + lax.axis_index("c")
    base = wid * ROWS_PER_W
    cols = ((a0_hbm, (a00, a01)), (a1_hbm, (a10, a11)), (a2_hbm, (a20, a21)))
    idxs = (idx_a, idx_b)
    sin = (si0, si1)
    sout = (so0, so1)
    sg = (sg0, sg1)

    @pl.when(sid == 0)
    def _build():
        pltpu.sync_copy(w0_hbm, w0_v)
        pltpu.sync_copy(w1_hbm, w1_v)
        pltpu.sync_copy(w2_hbm, w2_v)

        def build_combo(c, _):
            i = c // 12
            r = c - i * 12
            j = r // 2
            k = r - j * 2
            for v in range(8):
                sl = pl.ds(16 * v, 16)
                combo_v[c, sl] = w0_v[i, sl] + w1_v[j, sl] + w2_v[k, sl]
            return _

        lax.fori_loop(0, 60, build_combo, None)
        pltpu.sync_copy(combo_v, combo_sh)

    plsc.subcore_barrier()

    def start_in(g, b):
        for hbm, v in cols:
            pltpu.async_copy(hbm.at[pl.ds(base + g * CB, CB)],
                             v[b], sin[b])

    def wait_in(b):
        for hbm, v in cols:
            pltpu.make_async_copy(hbm.at[pl.ds(0, CB)], v[b],
                                  sin[b]).wait()

    def start_out(g, b):
        pltpu.async_copy(out_v.at[b], out_hbm.at[pl.ds(base + g * CB, CB)],
                         sout[b])

    def wait_out(b):
        pltpu.make_async_copy(out_v.at[b], out_hbm.at[pl.ds(0, CB)],
                              sout[b]).wait()

    def comp_idx(b):
        @plsc.parallel_loop(0, CB // 16)
        def _t(t):
            sl = pl.ds(16 * t, 16)
            idxs[b][sl] = (cols[0][1][b][sl] * 12 + cols[1][1][b][sl] * 2
                           + cols[2][1][b][sl])

    def start_gather(b):
        pltpu.async_copy(combo_sh.at[idxs[b]], out_v.at[b], sg[b])

    def wait_gather(b):
        pltpu.make_async_copy(combo_sh.at[idxs[b]], out_v.at[b],
                              sg[b]).wait()

    start_in(0, 0)
    start_in(1, 1)
    wait_in(0)
    comp_idx(0)
    start_gather(0)

    def pair_body(p, _):
        wait_in(1)
        comp_idx(1)

        @pl.when(p >= 1)
        def _():
            wait_out(1)

        start_gather(1)
        wait_gather(0)
        start_out(2 * p, 0)
        start_in(2 * p + 2, 0)

        wait_in(0)
        comp_idx(0)
        wait_out(0)
        start_gather(0)
        wait_gather(1)
        start_out(2 * p + 1, 1)

        @pl.when(p < 11)
        def _():
            start_in(2 * p + 3, 1)
        return _

    lax.fori_loop(0, (NCHUNK - 1) // 2, pair_body, None)

    wait_gather(0)
    start_out(NCHUNK - 1, 0)
    wait_out(1)
    wait_out(0)


def kernel(edge_attr, W0, W1, W2):
    ea = edge_attr.astype(jnp.int32)
    return _bond_encoder_sc(ea[:, 0], ea[:, 1], ea[:, 2], W0, W1, W2)


# speedup vs baseline: 19.4438x; 1.0291x over previous
import functools

import jax
import jax.numpy as jnp
from jax import lax
from jax.experimental import pallas as pl
from jax.experimental.pallas import tpu as pltpu
from jax.experimental.pallas import tpu_sc as plsc

E = 320000
D = 128
NC = 2
NS = 16
NW = NC * NS
ROWS_PER_W = E // NW
CB = 400
NCHUNK = ROWS_PER_W // CB

_mesh = plsc.VectorSubcoreMesh(core_axis_name="c", subcore_axis_name="s")


@functools.partial(
    pl.kernel,
    out_type=jax.ShapeDtypeStruct((E, D), jnp.float32),
    mesh=_mesh,
    scratch_types=[
        pltpu.VMEM((5, D), jnp.float32),
        pltpu.VMEM((6, D), jnp.float32),
        pltpu.VMEM((2, D), jnp.float32),
        pltpu.VMEM((8, D), jnp.float32),
        pltpu.VMEM_SHARED((64, D), jnp.float32),
        pltpu.VMEM((CB,), jnp.int32),
        pltpu.VMEM((CB,), jnp.int32),
        pltpu.VMEM((CB,), jnp.int32),
        pltpu.VMEM((CB,), jnp.int32),
        pltpu.VMEM((CB,), jnp.int32),
        pltpu.VMEM((CB,), jnp.int32),
        pltpu.VMEM((CB,), jnp.int32),
        pltpu.VMEM((CB,), jnp.int32),
        pltpu.VMEM((2, CB, D), jnp.float32),
        pltpu.SemaphoreType.DMA,
        pltpu.SemaphoreType.DMA,
        pltpu.SemaphoreType.DMA,
        pltpu.SemaphoreType.DMA,
        pltpu.SemaphoreType.DMA,
        pltpu.SemaphoreType.DMA,
    ],
)
def _bond_encoder_sc(a0_hbm, a1_hbm, a2_hbm, w0_hbm, w1_hbm, w2_hbm, out_hbm,
                     w0_v, w1_v, w2_v, combo_v, combo_sh,
                     a00, a01, a10, a11, a20, a21, idx_a, idx_b, out_v,
                     si0, si1, so0, so1, sg0, sg1):
    sid = lax.axis_index("s")
    wid = sid * NC + lax.axis_index("c")
    base = wid * ROWS_PER_W
    cols = ((a0_hbm, (a00, a01)), (a1_hbm, (a10, a11)), (a2_hbm, (a20, a21)))
    idxs = (idx_a, idx_b)
    sin = (si0, si1)
    sout = (so0, so1)
    sg = (sg0, sg1)


    def start_in(g, b):
        for hbm, v in cols:
            pltpu.async_copy(hbm.at[pl.ds(base + g * CB, CB)],
                             v[b], sin[b])

    def wait_in(b):
        for hbm, v in cols:
            pltpu.make_async_copy(hbm.at[pl.ds(0, CB)], v[b],
                                  sin[b]).wait()

    def start_out(g, b):
        pltpu.async_copy(out_v.at[b], out_hbm.at[pl.ds(base + g * CB, CB)],
                         sout[b])

    def wait_out(b):
        pltpu.make_async_copy(out_v.at[b], out_hbm.at[pl.ds(0, CB)],
                              sout[b]).wait()

    def comp_idx(b):
        @plsc.parallel_loop(0, CB // 16)
        def _t(t):
            sl = pl.ds(16 * t, 16)
            idxs[b][sl] = (cols[0][1][b][sl] * 12 + cols[1][1][b][sl] * 2
                           + cols[2][1][b][sl])

    def start_gather(b):
        pltpu.async_copy(combo_sh.at[idxs[b]], out_v.at[b], sg[b])

    def wait_gather(b):
        pltpu.make_async_copy(combo_sh.at[idxs[b]], out_v.at[b],
                              sg[b]).wait()

    start_in(0, 0)
    start_in(1, 1)

    @pl.when(sid < 8)
    def _build():
        pltpu.sync_copy(w0_hbm, w0_v)
        pltpu.sync_copy(w1_hbm, w1_v)
        pltpu.sync_copy(w2_hbm, w2_v)

        def build_combo(r, _):
            c = jnp.minimum(8 * sid + r, 59)
            i = c // 12
            rr = c - i * 12
            j = rr // 2
            k = rr - j * 2
            for v in range(8):
                sl = pl.ds(16 * v, 16)
                combo_v[r, sl] = w0_v[i, sl] + w1_v[j, sl] + w2_v[k, sl]
            return _

        lax.fori_loop(0, 8, build_combo, None)
        pltpu.sync_copy(combo_v, combo_sh.at[pl.ds(8 * sid, 8)])

    wait_in(0)
    comp_idx(0)
    plsc.subcore_barrier()
    start_gather(0)

    def pair_body(p, _):
        wait_in(1)
        comp_idx(1)

        @pl.when(p >= 1)
        def _():
            wait_out(1)

        start_gather(1)
        wait_gather(0)
        start_out(2 * p, 0)
        start_in(2 * p + 2, 0)

        wait_in(0)
        comp_idx(0)
        wait_out(0)
        start_gather(0)
        wait_gather(1)
        start_out(2 * p + 1, 1)

        @pl.when(p < 11)
        def _():
            start_in(2 * p + 3, 1)
        return _

    lax.fori_loop(0, (NCHUNK - 1) // 2, pair_body, None)

    wait_gather(0)
    start_out(NCHUNK - 1, 0)
    wait_out(1)
    wait_out(0)


def kernel(edge_attr, W0, W1, W2):
    ea = edge_attr.astype(jnp.int32)
    return _bond_encoder_sc(ea[:, 0], ea[:, 1], ea[:, 2], W0, W1, W2)
